# Initial kernel scaffold; baseline (speedup 1.0000x reference)
import jax
import jax.numpy as jnp
from jax.experimental import pallas as pl


def kernel(x, weight):
    raise NotImplementedError("write your pallas kernel here")


# speedup vs baseline: 1.5464x; 1.5464x over previous
import functools

import jax
import jax.numpy as jnp
from jax import lax
from jax.experimental import pallas as pl
from jax.experimental.pallas import tpu as pltpu
from jax.experimental.pallas import tpu_sc as plsc

VOCAB = 1000000
EMBED_DIM = 32
BATCH = 16384
NUM_FIELDS = 26
TOTAL = BATCH * NUM_FIELDS

NUM_WORKERS = 32
PER_WORKER = TOTAL // NUM_WORKERS
CHUNK = 1024
NUM_CHUNKS = PER_WORKER // CHUNK


@jax.jit
def _lookup(x_flat, weight):
    mesh = plsc.VectorSubcoreMesh(core_axis_name="c", subcore_axis_name="s")

    @functools.partial(
        pl.kernel,
        mesh=mesh,
        out_type=jax.ShapeDtypeStruct((TOTAL, EMBED_DIM), jnp.float32),
        scratch_types=[
            pltpu.VMEM((CHUNK,), jnp.int32),
            pltpu.VMEM((CHUNK, EMBED_DIM), jnp.float32),
            pltpu.SemaphoreType.DMA,
        ],
        compiler_params=pltpu.CompilerParams(use_tc_tiling_on_sc=False),
    )
    def body(idx_hbm, table_hbm, out_hbm, idx_v, rows_v, sem):
        wid = lax.axis_index("s") * 2 + lax.axis_index("c")
        base = wid * PER_WORKER

        def step(i, _):
            off = base + i * CHUNK
            pltpu.sync_copy(idx_hbm.at[pl.ds(off, CHUNK)], idx_v)
            pltpu.async_copy(table_hbm.at[idx_v], rows_v, sem).wait()
            pltpu.sync_copy(rows_v, out_hbm.at[pl.ds(off, CHUNK)])
            return _

        lax.fori_loop(0, NUM_CHUNKS, step, None)

    return body(x_flat, weight)


def kernel(x, weight):
    x_flat = x.reshape(TOTAL).astype(jnp.int32)
    out = _lookup(x_flat, weight)
    return out.reshape(BATCH, NUM_FIELDS, EMBED_DIM)


# speedup vs baseline: 1.5749x; 1.0185x over previous
import functools

import jax
import jax.numpy as jnp
from jax import lax
from jax.experimental import pallas as pl
from jax.experimental.pallas import tpu as pltpu
from jax.experimental.pallas import tpu_sc as plsc

VOCAB = 1000000
EMBED_DIM = 32
BATCH = 16384
NUM_FIELDS = 26
TOTAL = BATCH * NUM_FIELDS

NUM_WORKERS = 32
PER_WORKER = TOTAL // NUM_WORKERS
CHUNK = 832
NUM_CHUNKS = PER_WORKER // CHUNK
NBUF = 4


@jax.jit
def _lookup(x_flat, weight):
    mesh = plsc.VectorSubcoreMesh(core_axis_name="c", subcore_axis_name="s")

    @functools.partial(
        pl.kernel,
        mesh=mesh,
        out_type=jax.ShapeDtypeStruct((TOTAL, EMBED_DIM), jnp.float32),
        scratch_types=[
            pltpu.VMEM((PER_WORKER,), jnp.int32),
            pltpu.VMEM((NBUF, CHUNK, EMBED_DIM), jnp.float32),
            pltpu.SemaphoreType.DMA((NBUF,)),
            pltpu.SemaphoreType.DMA((NBUF,)),
        ],
        compiler_params=pltpu.CompilerParams(use_tc_tiling_on_sc=False),
    )
    def body(idx_hbm, table_hbm, out_hbm, idx_v, rows_v, gsem, ssem):
        wid = lax.axis_index("s") * 2 + lax.axis_index("c")
        base = wid * PER_WORKER
        pltpu.sync_copy(idx_hbm.at[pl.ds(base, PER_WORKER)], idx_v)

        def start_gather(i, slot):
            pltpu.async_copy(
                table_hbm.at[idx_v.at[pl.ds(i * CHUNK, CHUNK)]],
                rows_v.at[slot],
                gsem.at[slot],
            )

        def wait_gather(i, slot):
            pltpu.make_async_copy(
                table_hbm.at[idx_v.at[pl.ds(i * CHUNK, CHUNK)]],
                rows_v.at[slot],
                gsem.at[slot],
            ).wait()

        def start_store(i, slot):
            pltpu.async_copy(
                rows_v.at[slot],
                out_hbm.at[pl.ds(base + i * CHUNK, CHUNK)],
                ssem.at[slot],
            )

        def wait_store(i, slot):
            pltpu.make_async_copy(
                rows_v.at[slot],
                out_hbm.at[pl.ds(base + i * CHUNK, CHUNK)],
                ssem.at[slot],
            ).wait()

        start_gather(0, 0)
        start_gather(1, 1)

        for i in (0, 1):
            start_gather(i + 2, i + 2)
            wait_gather(i, i)
            start_store(i, i)

        def main(t, _):
            i0 = 2 + t * NBUF
            for b in range(NBUF):
                i = i0 + b
                wait_store(i - 2, b)
                start_gather(i + 2, b)
                wait_gather(i, (2 + b) % NBUF)
                start_store(i, (2 + b) % NBUF)
            return _

        lax.fori_loop(0, (NUM_CHUNKS - 4) // NBUF, main, None)

        for i in (NUM_CHUNKS - 2, NUM_CHUNKS - 1):
            wait_gather(i, i % NBUF)
            start_store(i, i % NBUF)

        for i in range(NUM_CHUNKS - NBUF, NUM_CHUNKS):
            wait_store(i, i % NBUF)

    return body(x_flat, weight)


def kernel(x, weight):
    x_flat = x.reshape(TOTAL).astype(jnp.int32)
    out = _lookup(x_flat, weight)
    return out.reshape(BATCH, NUM_FIELDS, EMBED_DIM)
